# Initial kernel scaffold; baseline (speedup 1.0000x reference)
import functools
import jax
import jax.numpy as jnp
from jax import lax
from jax.experimental import pallas as pl
from jax.experimental.pallas import tpu as pltpu
from jax.experimental.pallas import tpu_sc as plsc

_N = 10000
_E = 160000
_D = 16
_NET = 4
_FIN = 128

_NPAD = 10240
_ROWS_PER_SUB = _NPAD // 16
_CHUNK = 128
_NSUB = 32
_EPAD = 163840
_EDGES_PER_W = _EPAD // _NSUB
_NCHUNK = _EDGES_PER_W // _CHUNK

_BN = 512
_GRID = _NPAD // _BN


def _node_body(x_ref, w0_ref, b0_ref, ma_ref, map_ref, mb_ref, mbr_ref,
               bu_ref, bi_ref, u_ref, h_ref, tg_ref):
    h = jnp.maximum(x_ref[...] @ w0_ref[...] + b0_ref[...], 0.0)
    u = u_ref[...]
    A = h @ ma_ref[...]
    Ap = h @ map_ref[...]
    u64 = jnp.concatenate([u, u, u, u], axis=1)
    Tm = A + u64 * Ap
    C = h @ mb_ref[...]
    Cr = h @ mbr_ref[...]
    B = jnp.zeros_like(h)
    Br = jnp.zeros_like(h)
    for k in range(_D):
        hk = h[:, k:k + 1]
        B = B + hk * C[:, k * _D:(k + 1) * _D]
        Br = Br + hk * Cr[:, k * _D:(k + 1) * _D]
    bias = u * (h @ bu_ref[...]) + (1.0 - u) * (h @ bi_ref[...])
    Gm = B + bias + (1.0 - u) * Br
    h_ref[...] = h
    tg_ref[...] = jnp.concatenate([Tm, Gm], axis=1)


def _node_precompute(x_pad, W0, b0, M_A, M_Ap, M_B, M_Br, bu, bi, u16):
    full = lambda i: (0, 0)
    return pl.pallas_call(
        _node_body,
        grid=(_GRID,),
        in_specs=[
            pl.BlockSpec((_BN, _FIN), lambda i: (i, 0)),
            pl.BlockSpec((_FIN, _D), full),
            pl.BlockSpec((1, _D), full),
            pl.BlockSpec((_D, _NET * _D), full),
            pl.BlockSpec((_D, _NET * _D), full),
            pl.BlockSpec((_D, _D * _D), full),
            pl.BlockSpec((_D, _D * _D), full),
            pl.BlockSpec((_D, _D), full),
            pl.BlockSpec((_D, _D), full),
            pl.BlockSpec((_BN, _D), lambda i: (i, 0)),
        ],
        out_specs=[
            pl.BlockSpec((_BN, _D), lambda i: (i, 0)),
            pl.BlockSpec((_BN, 80), lambda i: (i, 0)),
        ],
        out_shape=[
            jax.ShapeDtypeStruct((_NPAD, _D), jnp.float32),
            jax.ShapeDtypeStruct((_NPAD, 80), jnp.float32),
        ],
    )(x_pad, W0, b0, M_A, M_Ap, M_B, M_Br, bu, bi, u16)


def _edge_body(tg_hbm, src_hbm, dst_hbm, ea_hbm, zeros_hbm, out_hbm,
               src_v, dst_v, ea_v, rows_v, msg_v, agg_sh, sem):
    cid = lax.axis_index("c")
    sid = lax.axis_index("s")
    r0 = sid * _ROWS_PER_SUB
    pltpu.sync_copy(zeros_hbm.at[pl.ds(r0, _ROWS_PER_SUB)],
                    agg_sh.at[pl.ds(r0, _ROWS_PER_SUB)])
    cnt_pat = jnp.where(lax.iota(jnp.int32, 16) == 0,
                        jnp.float32(1.0), jnp.float32(0.0))

    def _pref(j, carry):
        msg_v[j, pl.ds(16, 16)] = cnt_pat
        return carry

    lax.fori_loop(0, _CHUNK, _pref, 0)
    plsc.subcore_barrier()

    base = (cid * 16 + sid) * _EDGES_PER_W

    def _chunk(c, carry):
        eb = base + c * _CHUNK
        pltpu.sync_copy(src_hbm.at[pl.ds(eb, _CHUNK)], src_v)
        pltpu.sync_copy(dst_hbm.at[pl.ds(eb, _CHUNK)], dst_v)
        pltpu.sync_copy(ea_hbm.at[pl.ds(eb, _CHUNK)], ea_v)
        pltpu.async_copy(tg_hbm.at[src_v], rows_v, sem).wait()

        def _edge(j, inner):
            acc = rows_v[j, pl.ds(64, 16)]
            acc = acc + ea_v[j, 0] * rows_v[j, pl.ds(0, 16)]
            acc = acc + ea_v[j, 1] * rows_v[j, pl.ds(16, 16)]
            acc = acc + ea_v[j, 2] * rows_v[j, pl.ds(32, 16)]
            acc = acc + ea_v[j, 3] * rows_v[j, pl.ds(48, 16)]
            msg_v[j, pl.ds(0, 16)] = acc
            return inner

        lax.fori_loop(0, _CHUNK, _edge, 0)
        pltpu.sync_copy(msg_v, agg_sh.at[dst_v], add=True)
        return carry

    lax.fori_loop(0, _NCHUNK, _chunk, 0)
    plsc.subcore_barrier()
    pltpu.sync_copy(agg_sh.at[pl.ds(r0, _ROWS_PER_SUB)],
                    out_hbm.at[cid, pl.ds(r0, _ROWS_PER_SUB)])


def _edge_phase(tg, src_pad, dst_pad, ea_pad, zeros32):
    mesh = plsc.VectorSubcoreMesh(core_axis_name="c", subcore_axis_name="s")
    kern = pl.kernel(
        _edge_body,
        out_type=jax.ShapeDtypeStruct((2, _NPAD, 32), jnp.float32),
        mesh=mesh,
        scratch_types=[
            pltpu.VMEM((_CHUNK,), jnp.int32),
            pltpu.VMEM((_CHUNK,), jnp.int32),
            pltpu.VMEM((_CHUNK, _NET), jnp.float32),
            pltpu.VMEM((_CHUNK, 80), jnp.float32),
            pltpu.VMEM((_CHUNK, 32), jnp.float32),
            pltpu.VMEM_SHARED((_NPAD, 32), jnp.float32),
            pltpu.SemaphoreType.DMA,
        ],
    )
    return kern(tg, src_pad, dst_pad, ea_pad, zeros32)


def _final_body(p0_ref, p1_ref, h_ref, wih_ref, whh_ref, bih_ref, bhh_ref,
                out_ref):
    s = p0_ref[...] + p1_ref[...]
    agg = s[:, 0:16]
    cnt = s[:, 16:17]
    mean = agg / jnp.maximum(cnt, 1.0)
    m = jnp.maximum(mean, 0.0)
    h = h_ref[...]
    gi = m @ wih_ref[...] + bih_ref[...]
    gh = h @ whh_ref[...] + bhh_ref[...]
    r = jax.nn.sigmoid(gi[:, 0:16] + gh[:, 0:16])
    z = jax.nn.sigmoid(gi[:, 16:32] + gh[:, 16:32])
    n = jnp.tanh(gi[:, 32:48] + r * gh[:, 32:48])
    h_new = (1.0 - z) * n + z * h
    out_ref[...] = jnp.maximum(h_new, 0.0)


def _finalize(p0, p1, h, Wih, Whh, bih, bhh):
    full = lambda i: (0, 0)
    return pl.pallas_call(
        _final_body,
        grid=(_GRID,),
        in_specs=[
            pl.BlockSpec((_BN, 32), lambda i: (i, 0)),
            pl.BlockSpec((_BN, 32), lambda i: (i, 0)),
            pl.BlockSpec((_BN, _D), lambda i: (i, 0)),
            pl.BlockSpec((_D, 3 * _D), full),
            pl.BlockSpec((_D, 3 * _D), full),
            pl.BlockSpec((1, 3 * _D), full),
            pl.BlockSpec((1, 3 * _D), full),
        ],
        out_specs=pl.BlockSpec((_BN, _D), lambda i: (i, 0)),
        out_shape=jax.ShapeDtypeStruct((_NPAD, _D), jnp.float32),
    )(p0, p1, h, Wih, Whh, bih, bhh)


def kernel(x, edge_index, edge_attr, is_user, W0, b0, Wr, br, Wp, bp,
           We, be, Wih, Whh, bih, bhh):
    f32 = jnp.float32
    We3a = We[:_NET].reshape(_NET, _D, _D)
    We3x = We[_NET:].reshape(_D, _D, _D)
    Wp3 = Wp.reshape(_NET, _D, _D)
    Wr3 = Wr.reshape(_D, _D, _D)
    M_A = We3a.transpose(1, 0, 2).reshape(_D, _NET * _D)
    M_Ap = Wp3.transpose(1, 0, 2).reshape(_D, _NET * _D)
    M_B = We3x.transpose(1, 0, 2).reshape(_D, _D * _D)
    M_Br = Wr3.transpose(1, 0, 2).reshape(_D, _D * _D)
    bu = (be + bp).reshape(_D, _D)
    bi = (be + br).reshape(_D, _D)

    x_pad = jnp.zeros((_NPAD, _FIN), f32).at[:_N].set(x)
    u16 = jnp.zeros((_NPAD, _D), f32).at[:_N].set(
        jnp.broadcast_to(is_user.astype(f32)[:, None], (_N, _D)))
    src_pad = jnp.zeros((_EPAD,), jnp.int32).at[:_E].set(edge_index[0])
    dst_pad = jnp.full((_EPAD,), _N, jnp.int32).at[:_E].set(edge_index[1])
    ea_pad = jnp.zeros((_EPAD, _NET), f32).at[:_E].set(edge_attr)
    zeros32 = jnp.zeros((_NPAD, 32), f32)

    h, tg = _node_precompute(x_pad, W0, b0.reshape(1, _D), M_A, M_Ap,
                             M_B, M_Br, bu, bi, u16)
    partial = _edge_phase(tg, src_pad, dst_pad, ea_pad, zeros32)
    out = _finalize(partial[0], partial[1], h, Wih, Whh,
                    bih.reshape(1, 3 * _D), bhh.reshape(1, 3 * _D))
    return out[:_N]


# speedup vs baseline: 6.8956x; 6.8956x over previous
import functools
import jax
import jax.numpy as jnp
from jax import lax
from jax.experimental import pallas as pl
from jax.experimental.pallas import tpu as pltpu
from jax.experimental.pallas import tpu_sc as plsc

_N = 10000
_E = 160000
_D = 16
_NET = 4
_FIN = 128

_NPAD = 10240
_ROWS_PER_SUB = _NPAD // 16
_CHUNK = 128
_NCORES = 2
_NSUB = 16 * _NCORES
_EPAD = 163840
_EDGES_PER_W = _EPAD // _NSUB
_NCHUNK = _EDGES_PER_W // _CHUNK
_NCHUNKP = _NCHUNK + 4

_BN = 512
_GRID = _NPAD // _BN


def _node_body(x_ref, w0_ref, b0_ref, ma_ref, map_ref, mb_ref, mbr_ref,
               bu_ref, bi_ref, rep_ref, sum_ref, u_ref, h_ref, tg_ref):
    h = jnp.maximum(x_ref[...] @ w0_ref[...] + b0_ref[...], 0.0)
    u = u_ref[...]
    A = h @ ma_ref[...]
    Ap = h @ map_ref[...]
    u64 = jnp.concatenate([u, u, u, u], axis=1)
    Tm = A + u64 * Ap
    C = h @ mb_ref[...]
    Cr = h @ mbr_ref[...]
    hrep = h @ rep_ref[...]
    B = (C * hrep) @ sum_ref[...]
    Br = (Cr * hrep) @ sum_ref[...]
    bias = u * (h @ bu_ref[...]) + (1.0 - u) * (h @ bi_ref[...])
    Gm = B + bias + (1.0 - u) * Br
    h_ref[...] = h
    tg_ref[...] = jnp.concatenate([Tm, Gm], axis=1)


def _node_precompute(x_pad, W0, b0, M_A, M_Ap, M_B, M_Br, bu, bi,
                     rep_m, sum_m, u16):
    full = lambda i: (0, 0)
    return pl.pallas_call(
        _node_body,
        grid=(_GRID,),
        in_specs=[
            pl.BlockSpec((_BN, _FIN), lambda i: (i, 0)),
            pl.BlockSpec((_FIN, _D), full),
            pl.BlockSpec((1, _D), full),
            pl.BlockSpec((_D, _NET * _D), full),
            pl.BlockSpec((_D, _NET * _D), full),
            pl.BlockSpec((_D, _D * _D), full),
            pl.BlockSpec((_D, _D * _D), full),
            pl.BlockSpec((_D, _D), full),
            pl.BlockSpec((_D, _D), full),
            pl.BlockSpec((_D, _D * _D), full),
            pl.BlockSpec((_D * _D, _D), full),
            pl.BlockSpec((_BN, _D), lambda i: (i, 0)),
        ],
        out_specs=[
            pl.BlockSpec((_BN, _D), lambda i: (i, 0)),
            pl.BlockSpec((_BN, 80), lambda i: (i, 0)),
        ],
        out_shape=[
            jax.ShapeDtypeStruct((_NPAD, _D), jnp.float32),
            jax.ShapeDtypeStruct((_NPAD, 80), jnp.float32),
        ],
    )(x_pad, W0, b0, M_A, M_Ap, M_B, M_Br, bu, bi, rep_m, sum_m, u16)


def _edge_body(tg_hbm, src_hbm, dst_hbm, ea_hbm, zeros_hbm, out_hbm,
               src_v, dst_v, ea0_v, ea1_v, rows0_v, rows1_v, msg0_v, msg1_v,
               dump_v, tgb_v, bounce_v, tg_sh, agg_sh,
               sem_g0, sem_g1, sem_e0, sem_e1, sem_s0, sem_s1):
    cid = lax.axis_index("c")
    sid = lax.axis_index("s")
    wid = cid * 16 + sid
    r0 = sid * _ROWS_PER_SUB
    for k in range(_ROWS_PER_SUB // _CHUNK):
        rk = r0 + k * _CHUNK
        pltpu.sync_copy(zeros_hbm.at[pl.ds(rk, _CHUNK)], bounce_v)
        pltpu.sync_copy(bounce_v, agg_sh.at[pl.ds(rk, _CHUNK)])
    for k in range(_ROWS_PER_SUB // _CHUNK):
        rk = r0 + k * _CHUNK
        pltpu.sync_copy(tg_hbm.at[pl.ds(rk, _CHUNK)], tgb_v)
        pltpu.sync_copy(tgb_v, tg_sh.at[pl.ds(rk, _CHUNK)])
    pltpu.sync_copy(src_hbm.at[wid], src_v)
    pltpu.sync_copy(dst_hbm.at[wid], dst_v)
    cnt_pat = jnp.where(lax.iota(jnp.int32, 16) == 0,
                        jnp.float32(1.0), jnp.float32(0.0))
    zero16 = jnp.zeros((16,), jnp.float32)

    def _pref(j, carry):
        msg0_v[j, pl.ds(0, 16)] = zero16
        msg0_v[j, pl.ds(16, 16)] = cnt_pat
        msg1_v[j, pl.ds(0, 16)] = zero16
        msg1_v[j, pl.ds(16, 16)] = cnt_pat
        return carry

    lax.fori_loop(0, _CHUNK, _pref, 0)
    dump_pat = jnp.full((16,), _N, jnp.int32)
    for k in range(_CHUNK // 16):
        dump_v[pl.ds(k * 16, 16)] = dump_pat
    plsc.subcore_barrier()

    pltpu.async_copy(tg_sh.at[src_v.at[0]], rows0_v, sem_g0)
    pltpu.async_copy(ea_hbm.at[wid, 0], ea0_v.at[pl.ds(0, _CHUNK * _NET)], sem_e0)
    pltpu.async_copy(tg_sh.at[src_v.at[1]], rows1_v, sem_g1)
    pltpu.async_copy(ea_hbm.at[wid, 1], ea1_v.at[pl.ds(0, _CHUNK * _NET)], sem_e1)
    pltpu.async_copy(msg0_v, agg_sh.at[dump_v], sem_s0, add=True)
    pltpu.async_copy(msg1_v, agg_sh.at[dump_v], sem_s1, add=True)

    def _do_chunk(c, rows_v, ea_v, msg_v, sem_g, sem_e, sem_s):
        pltpu.make_async_copy(tg_hbm.at[pl.ds(0, _CHUNK)], rows_v, sem_g).wait()
        pltpu.make_async_copy(ea_hbm.at[wid, 0], ea_v.at[pl.ds(0, _CHUNK * _NET)], sem_e).wait()
        pltpu.make_async_copy(msg_v, agg_sh.at[dump_v], sem_s).wait()

        def _edge(j, inner):
            ev = ea_v[pl.ds(j * _NET, 16)]
            acc = rows_v[j, pl.ds(64, 16)]
            acc = acc + ev[0] * rows_v[j, pl.ds(0, 16)]
            acc = acc + ev[1] * rows_v[j, pl.ds(16, 16)]
            acc = acc + ev[2] * rows_v[j, pl.ds(32, 16)]
            acc = acc + ev[3] * rows_v[j, pl.ds(48, 16)]
            msg_v[j, pl.ds(0, 16)] = acc
            return inner

        lax.fori_loop(0, _CHUNK, _edge, 0)
        pltpu.async_copy(msg_v, agg_sh.at[dst_v.at[c]], sem_s, add=True)
        pltpu.async_copy(tg_sh.at[src_v.at[c + 2]], rows_v, sem_g)
        pltpu.async_copy(ea_hbm.at[wid, c + 2], ea_v.at[pl.ds(0, _CHUNK * _NET)], sem_e)

    def _pair(p, carry):
        _do_chunk(2 * p, rows0_v, ea0_v, msg0_v, sem_g0, sem_e0, sem_s0)
        _do_chunk(2 * p + 1, rows1_v, ea1_v, msg1_v, sem_g1, sem_e1, sem_s1)
        return carry

    lax.fori_loop(0, _NCHUNK // 2, _pair, 0)
    pltpu.make_async_copy(tg_hbm.at[pl.ds(0, _CHUNK)], rows0_v, sem_g0).wait()
    pltpu.make_async_copy(ea_hbm.at[wid, 0], ea0_v.at[pl.ds(0, _CHUNK * _NET)], sem_e0).wait()
    pltpu.make_async_copy(msg0_v, agg_sh.at[dump_v], sem_s0).wait()
    pltpu.make_async_copy(tg_hbm.at[pl.ds(0, _CHUNK)], rows1_v, sem_g1).wait()
    pltpu.make_async_copy(ea_hbm.at[wid, 0], ea1_v.at[pl.ds(0, _CHUNK * _NET)], sem_e1).wait()
    pltpu.make_async_copy(msg1_v, agg_sh.at[dump_v], sem_s1).wait()
    plsc.subcore_barrier()
    for k in range(_ROWS_PER_SUB // _CHUNK):
        rk = r0 + k * _CHUNK
        pltpu.sync_copy(agg_sh.at[pl.ds(rk, _CHUNK)], bounce_v)
        pltpu.sync_copy(bounce_v, out_hbm.at[cid, pl.ds(rk, _CHUNK)])


def _edge_phase(tg, src3, dst3, ea4, zeros32):
    mesh = plsc.VectorSubcoreMesh(core_axis_name="c", subcore_axis_name="s", num_cores=_NCORES)
    kern = pl.kernel(
        _edge_body,
        out_type=jax.ShapeDtypeStruct((_NCORES, _NPAD, 32), jnp.float32),
        mesh=mesh,
        scratch_types=[
            pltpu.VMEM((_NCHUNKP, _CHUNK), jnp.int32),
            pltpu.VMEM((_NCHUNKP, _CHUNK), jnp.int32),
            pltpu.VMEM((_CHUNK * _NET + 16,), jnp.float32),
            pltpu.VMEM((_CHUNK * _NET + 16,), jnp.float32),
            pltpu.VMEM((_CHUNK, 80), jnp.float32),
            pltpu.VMEM((_CHUNK, 80), jnp.float32),
            pltpu.VMEM((_CHUNK, 32), jnp.float32),
            pltpu.VMEM((_CHUNK, 32), jnp.float32),
            pltpu.VMEM((_CHUNK,), jnp.int32),
            pltpu.VMEM((_CHUNK, 80), jnp.float32),
            pltpu.VMEM((_CHUNK, 32), jnp.float32),
            pltpu.VMEM_SHARED((_NPAD, 80), jnp.float32),
            pltpu.VMEM_SHARED((_NPAD, 32), jnp.float32),
            pltpu.SemaphoreType.DMA,
            pltpu.SemaphoreType.DMA,
            pltpu.SemaphoreType.DMA,
            pltpu.SemaphoreType.DMA,
            pltpu.SemaphoreType.DMA,
            pltpu.SemaphoreType.DMA,
        ],
        compiler_params=pltpu.CompilerParams(use_tc_tiling_on_sc=False),
    )
    return kern(tg, src3, dst3, ea4, zeros32)


def _final_body(p0_ref, p1_ref, h_ref, wih_ref, whh_ref, bih_ref, bhh_ref,
                out_ref):
    s = p0_ref[...] + p1_ref[...]
    agg = s[:, 0:16]
    cnt = s[:, 16:17]
    mean = agg / jnp.maximum(cnt, 1.0)
    m = jnp.maximum(mean, 0.0)
    h = h_ref[...]
    gi = m @ wih_ref[...] + bih_ref[...]
    gh = h @ whh_ref[...] + bhh_ref[...]
    r = jax.nn.sigmoid(gi[:, 0:16] + gh[:, 0:16])
    z = jax.nn.sigmoid(gi[:, 16:32] + gh[:, 16:32])
    n = jnp.tanh(gi[:, 32:48] + r * gh[:, 32:48])
    h_new = (1.0 - z) * n + z * h
    out_ref[...] = jnp.maximum(h_new, 0.0)


def _finalize(p0, p1, h, Wih, Whh, bih, bhh):
    full = lambda i: (0, 0)
    return pl.pallas_call(
        _final_body,
        grid=(_GRID,),
        in_specs=[
            pl.BlockSpec((_BN, 32), lambda i: (i, 0)),
            pl.BlockSpec((_BN, 32), lambda i: (i, 0)),
            pl.BlockSpec((_BN, _D), lambda i: (i, 0)),
            pl.BlockSpec((_D, 3 * _D), full),
            pl.BlockSpec((_D, 3 * _D), full),
            pl.BlockSpec((1, 3 * _D), full),
            pl.BlockSpec((1, 3 * _D), full),
        ],
        out_specs=pl.BlockSpec((_BN, _D), lambda i: (i, 0)),
        out_shape=jax.ShapeDtypeStruct((_NPAD, _D), jnp.float32),
    )(p0, p1, h, Wih, Whh, bih, bhh)


def kernel(x, edge_index, edge_attr, is_user, W0, b0, Wr, br, Wp, bp,
           We, be, Wih, Whh, bih, bhh):
    f32 = jnp.float32
    We3a = We[:_NET].reshape(_NET, _D, _D)
    We3x = We[_NET:].reshape(_D, _D, _D)
    Wp3 = Wp.reshape(_NET, _D, _D)
    Wr3 = Wr.reshape(_D, _D, _D)
    M_A = We3a.transpose(1, 0, 2).reshape(_D, _NET * _D)
    M_Ap = Wp3.transpose(1, 0, 2).reshape(_D, _NET * _D)
    M_B = We3x.transpose(1, 0, 2).reshape(_D, _D * _D)
    M_Br = Wr3.transpose(1, 0, 2).reshape(_D, _D * _D)
    bu = (be + bp).reshape(_D, _D)
    bi = (be + br).reshape(_D, _D)
    rep_m = jnp.kron(jnp.eye(_D, dtype=f32), jnp.ones((1, _D), f32))
    sum_m = jnp.kron(jnp.ones((_D, 1), f32), jnp.eye(_D, dtype=f32))

    x_pad = jnp.zeros((_NPAD, _FIN), f32).at[:_N].set(x)
    u16 = jnp.zeros((_NPAD, _D), f32).at[:_N].set(
        jnp.broadcast_to(is_user.astype(f32)[:, None], (_N, _D)))
    src_pad = jnp.zeros((_EPAD,), jnp.int32).at[:_E].set(edge_index[0])
    dst_pad = jnp.full((_EPAD,), _N, jnp.int32).at[:_E].set(edge_index[1])
    ea_pad = jnp.zeros((_EPAD, _NET), f32).at[:_E].set(edge_attr)
    src3 = jnp.zeros((_NSUB, _NCHUNKP, _CHUNK), jnp.int32)\
        .at[:, :_NCHUNK].set(src_pad.reshape(_NSUB, _NCHUNK, _CHUNK))
    dst3 = jnp.full((_NSUB, _NCHUNKP, _CHUNK), _N, jnp.int32)\
        .at[:, :_NCHUNK].set(dst_pad.reshape(_NSUB, _NCHUNK, _CHUNK))
    ea3 = jnp.zeros((_NSUB, _NCHUNKP, _CHUNK * _NET), f32)\
        .at[:, :_NCHUNK].set(ea_pad.reshape(_NSUB, _NCHUNK, _CHUNK * _NET))
    zeros32 = jnp.zeros((_NPAD, 32), f32)

    h, tg = _node_precompute(x_pad, W0, b0.reshape(1, _D), M_A, M_Ap,
                             M_B, M_Br, bu, bi, rep_m, sum_m, u16)
    partial = _edge_phase(tg, src3, dst3, ea3, zeros32)
    p1 = partial[1] if _NCORES == 2 else zeros32
    out = _finalize(partial[0], p1, h, Wih, Whh,
                    bih.reshape(1, 3 * _D), bhh.reshape(1, 3 * _D))
    return out[:_N]
